# baseline (speedup 1.0000x reference)
import functools

import jax
import jax.numpy as jnp
from jax import lax
from jax.experimental import pallas as pl
from jax.experimental.pallas import tpu as pltpu
from jax.experimental.pallas import tpu_sc as plsc

N_NODES = 10000
D = 128
DA = D + 16
N_EDGES = 320000

NC = 2
NS = 16
NW = NC * NS
CHUNK = 128
KCH = 80
GRP = 8
EPW = KCH * CHUNK
EPAD = NW * EPW
K0 = 32
K1 = 128
KMAX = max(K0, K1)
NPAD = 10240
RPS = NPAD // NS
SROWS = 128


def _make_agg(width):
    mesh = plsc.VectorSubcoreMesh(core_axis_name="c", subcore_axis_name="s")

    @functools.partial(
        pl.kernel, mesh=mesh,
        out_type=jax.ShapeDtypeStruct((NC, NPAD, width), jnp.float32),
        scratch_types=[
            pltpu.VMEM((GRP, CHUNK), jnp.int32),
            pltpu.VMEM((GRP, CHUNK), jnp.int32),
            pltpu.VMEM((CHUNK, width), jnp.float32),
            pltpu.VMEM((CHUNK, width), jnp.float32),
            pltpu.VMEM_SHARED((NPAD, width), jnp.float32),
            pltpu.SemaphoreType.DMA,
            pltpu.SemaphoreType.DMA,
        ])
    def k(table, srcs, dsts, zrows, agg_out, src_v, dst_v, rows_a, rows_b,
          agg_sh, sem_a, sem_b):
        rows_v = rows_a
        sid = lax.axis_index("s")
        cid = lax.axis_index("c")
        wid = sid * NC + cid

        pltpu.sync_copy(zrows, rows_v)
        for t in range(RPS // SROWS):
            pltpu.sync_copy(
                rows_v, agg_sh.at[pl.ds(sid * RPS + t * SROWS, SROWS)])
        plsc.subcore_barrier()

        bufs = (rows_a, rows_b)
        sems = (sem_a, sem_b)
        ngrp = jnp.where(cid == 0, K0 // GRP, K1 // GRP)

        def group(g, _):
            pltpu.sync_copy(srcs.at[wid].at[pl.ds(g * GRP, GRP)], src_v)
            pltpu.sync_copy(dsts.at[wid].at[pl.ds(g * GRP, GRP)], dst_v)

            pend = [None, None]
            pend[0] = pltpu.async_copy(
                table.at[src_v.at[0]], bufs[0], sems[0])
            for j in range(GRP):
                p = j % 2
                if j + 1 < GRP:
                    pend[1 - p] = pltpu.async_copy(
                        table.at[src_v.at[j + 1]], bufs[1 - p], sems[1 - p])
                pend[p].wait()
                pltpu.sync_copy(bufs[p], agg_sh.at[dst_v.at[j]], add=True)
            return 0

        lax.fori_loop(0, ngrp, group, 0)
        plsc.subcore_barrier()

        for t in range(RPS // SROWS):
            off = sid * RPS + t * SROWS
            pltpu.sync_copy(agg_sh.at[pl.ds(off, SROWS)], rows_v)
            pltpu.sync_copy(rows_v, agg_out.at[cid].at[pl.ds(off, SROWS)])

    return k


_agg_plain = _make_agg(D)


def _make_cnt():
    mesh = plsc.VectorSubcoreMesh(core_axis_name="c", subcore_axis_name="s")

    @functools.partial(
        pl.kernel, mesh=mesh,
        out_type=jax.ShapeDtypeStruct((NC * NPAD, D), jnp.float32),
        scratch_types=[
            pltpu.VMEM((GRP, CHUNK), jnp.int32),
            pltpu.VMEM((CHUNK, D), jnp.float32),
            pltpu.VMEM((SROWS, D), jnp.float32),
            pltpu.VMEM_SHARED((NPAD, D), jnp.float32),
        ])
    def k(dsts, ones_h, zc, cnt_out, dst_v, ones_v, stage, cnt_sh):
        sid = lax.axis_index("s")
        cid = lax.axis_index("c")
        wid = sid * NC + cid

        pltpu.sync_copy(zc, stage)
        for t in range(RPS // SROWS):
            pltpu.sync_copy(
                stage, cnt_sh.at[pl.ds(sid * RPS + t * SROWS, SROWS)])
        pltpu.sync_copy(ones_h, ones_v)
        plsc.subcore_barrier()

        def group(g, _):
            pltpu.sync_copy(dsts.at[wid].at[pl.ds(g * GRP, GRP)], dst_v)

            def chunk(j, _):
                pltpu.sync_copy(ones_v, cnt_sh.at[dst_v.at[j]], add=True)
                return 0

            lax.fori_loop(0, GRP, chunk, 0)
            return 0

        lax.fori_loop(0, KCH // GRP, group, 0)
        plsc.subcore_barrier()

        for t in range(RPS // SROWS):
            off = sid * RPS + t * SROWS
            pltpu.sync_copy(cnt_sh.at[pl.ds(off, SROWS)], stage)
            pltpu.sync_copy(stage, cnt_out.at[pl.ds(cid * NPAD + off, SROWS)])

    return k


_cnt_kernel = _make_cnt()


def _combine(agg, cnt, xin, wl_t, wr_t, b, relu):
    BLK = 2048
    grid = (5,)

    def body(agg_ref, cnt_ref, x_ref, wl_ref, wr_ref, b_ref, o_ref):
        aggs = agg_ref[0] + agg_ref[1]
        c = cnt_ref[0] + cnt_ref[1]
        inv = 1.0 / jnp.maximum(c, 1.0)
        a = aggs * inv[:, None]
        r = (jnp.dot(a, wl_ref[...], preferred_element_type=jnp.float32)
             + jnp.dot(x_ref[...], wr_ref[...],
                       preferred_element_type=jnp.float32)
             + b_ref[...])
        if relu:
            r = jnp.maximum(r, 0.0)
        o_ref[...] = r

    return pl.pallas_call(
        body,
        grid=grid,
        in_specs=[
            pl.BlockSpec((NC, BLK, D), lambda i: (0, i, 0)),
            pl.BlockSpec((NC, BLK), lambda i: (0, i)),
            pl.BlockSpec((BLK, D), lambda i: (i, 0)),
            pl.BlockSpec((D, D), lambda i: (0, 0)),
            pl.BlockSpec((D, D), lambda i: (0, 0)),
            pl.BlockSpec((1, D), lambda i: (0, 0)),
        ],
        out_specs=pl.BlockSpec((BLK, D), lambda i: (i, 0)),
        out_shape=jax.ShapeDtypeStruct((N_NODES, D), jnp.float32),
    )(agg, cnt, xin, wl_t, wr_t, b)


def kernel(x, edge_index, W1l, W1r, b1, W2l, W2r, b2):
    ei = edge_index.astype(jnp.int32)
    pad = EPAD - N_EDGES
    src = jnp.concatenate([ei[0], jnp.zeros((pad,), jnp.int32)])
    dst = jnp.concatenate(
        [ei[1], (jnp.arange(pad, dtype=jnp.int32) % 16) + N_NODES])
    dsts_cnt = dst.reshape(NW, KCH, CHUNK)

    kw = [K0 if (w % NC) == 0 else K1 for w in range(NW)]
    offs = [0]
    for w in range(NW - 1):
        offs.append(offs[-1] + kw[w])
    src_ch = src.reshape(EPAD // CHUNK, CHUNK)
    dst_ch = dst.reshape(EPAD // CHUNK, CHUNK)
    srcs = jnp.zeros((NW, KMAX, CHUNK), jnp.int32)
    dsts = jnp.full((NW, KMAX, CHUNK), N_NODES, jnp.int32)
    for w in range(NW):
        srcs = srcs.at[w, :kw[w]].set(src_ch[offs[w]:offs[w] + kw[w]])
        dsts = dsts.at[w, :kw[w]].set(dst_ch[offs[w]:offs[w] + kw[w]])
    zrows = jnp.zeros((SROWS, D), jnp.float32)

    ones_h = jnp.ones((CHUNK, D), jnp.float32)
    cntw = _cnt_kernel(dsts_cnt, ones_h, zrows)
    cnt = cntw.reshape(NC, NPAD, D)[:, :, 0]
    agg1 = _agg_plain(x, srcs, dsts, zrows)
    h = _combine(agg1, cnt, x, W1l.T, W1r.T, b1.reshape(1, D), relu=True)
    agg2 = _agg_plain(h, srcs, dsts, zrows)
    out = _combine(agg2, cnt, h, W2l.T, W2r.T, b2.reshape(1, D), relu=False)
    return out


# speedup vs baseline: 1.0351x; 1.0351x over previous
import functools

import jax
import jax.numpy as jnp
from jax import lax
from jax.experimental import pallas as pl
from jax.experimental.pallas import tpu as pltpu
from jax.experimental.pallas import tpu_sc as plsc

N_NODES = 10000
D = 128
DA = D + 16
N_EDGES = 320000

NC = 2
NS = 16
NW = NC * NS
CHUNK = 128
KCH = 80
GRP = 8
EPW = KCH * CHUNK
EPAD = NW * EPW
K0 = 160
K1 = 0
KMAX = max(K0, K1)
NPAD = 10240
RPS = NPAD // NS
SROWS = 128


def _make_agg(width):
    mesh = plsc.VectorSubcoreMesh(core_axis_name="c", subcore_axis_name="s")

    @functools.partial(
        pl.kernel, mesh=mesh,
        out_type=jax.ShapeDtypeStruct((NC, NPAD, width), jnp.float32),
        scratch_types=[
            pltpu.VMEM((GRP, CHUNK), jnp.int32),
            pltpu.VMEM((GRP, CHUNK), jnp.int32),
            pltpu.VMEM((CHUNK, width), jnp.float32),
            pltpu.VMEM((CHUNK, width), jnp.float32),
            pltpu.VMEM_SHARED((NPAD, width), jnp.float32),
            pltpu.SemaphoreType.DMA,
            pltpu.SemaphoreType.DMA,
        ])
    def k(table, srcs, dsts, zrows, agg_out, src_v, dst_v, rows_a, rows_b,
          agg_sh, sem_a, sem_b):
        rows_v = rows_a
        sid = lax.axis_index("s")
        cid = lax.axis_index("c")
        wid = sid * NC + cid

        pltpu.sync_copy(zrows, rows_v)
        for t in range(RPS // SROWS):
            pltpu.sync_copy(
                rows_v, agg_sh.at[pl.ds(sid * RPS + t * SROWS, SROWS)])
        plsc.subcore_barrier()

        bufs = (rows_a, rows_b)
        sems = (sem_a, sem_b)
        ngrp = jnp.where(cid == 0, K0 // GRP, K1 // GRP)

        def group(g, _):
            pltpu.sync_copy(srcs.at[wid].at[pl.ds(g * GRP, GRP)], src_v)
            pltpu.sync_copy(dsts.at[wid].at[pl.ds(g * GRP, GRP)], dst_v)

            pend = [None, None]
            pend[0] = pltpu.async_copy(
                table.at[src_v.at[0]], bufs[0], sems[0])
            for j in range(GRP):
                p = j % 2
                if j + 1 < GRP:
                    pend[1 - p] = pltpu.async_copy(
                        table.at[src_v.at[j + 1]], bufs[1 - p], sems[1 - p])
                pend[p].wait()
                pltpu.sync_copy(bufs[p], agg_sh.at[dst_v.at[j]], add=True)
            return 0

        lax.fori_loop(0, ngrp, group, 0)
        plsc.subcore_barrier()

        for t in range(RPS // SROWS):
            off = sid * RPS + t * SROWS
            pltpu.sync_copy(agg_sh.at[pl.ds(off, SROWS)], rows_v)
            pltpu.sync_copy(rows_v, agg_out.at[cid].at[pl.ds(off, SROWS)])

    return k


_agg_plain = _make_agg(D)


def _make_cnt():
    mesh = plsc.VectorSubcoreMesh(core_axis_name="c", subcore_axis_name="s")

    @functools.partial(
        pl.kernel, mesh=mesh,
        out_type=jax.ShapeDtypeStruct((NC * NPAD, D), jnp.float32),
        scratch_types=[
            pltpu.VMEM((GRP, CHUNK), jnp.int32),
            pltpu.VMEM((CHUNK, D), jnp.float32),
            pltpu.VMEM((SROWS, D), jnp.float32),
            pltpu.VMEM_SHARED((NPAD, D), jnp.float32),
        ])
    def k(dsts, ones_h, zc, cnt_out, dst_v, ones_v, stage, cnt_sh):
        sid = lax.axis_index("s")
        cid = lax.axis_index("c")
        wid = sid * NC + cid

        pltpu.sync_copy(zc, stage)
        for t in range(RPS // SROWS):
            pltpu.sync_copy(
                stage, cnt_sh.at[pl.ds(sid * RPS + t * SROWS, SROWS)])
        pltpu.sync_copy(ones_h, ones_v)
        plsc.subcore_barrier()

        def group(g, _):
            pltpu.sync_copy(dsts.at[wid].at[pl.ds(g * GRP, GRP)], dst_v)

            def chunk(j, _):
                pltpu.sync_copy(ones_v, cnt_sh.at[dst_v.at[j]], add=True)
                return 0

            lax.fori_loop(0, GRP, chunk, 0)
            return 0

        lax.fori_loop(0, KCH // GRP, group, 0)
        plsc.subcore_barrier()

        for t in range(RPS // SROWS):
            off = sid * RPS + t * SROWS
            pltpu.sync_copy(cnt_sh.at[pl.ds(off, SROWS)], stage)
            pltpu.sync_copy(stage, cnt_out.at[pl.ds(cid * NPAD + off, SROWS)])

    return k


_cnt_kernel = _make_cnt()


def _combine(agg, cnt, xin, wl_t, wr_t, b, relu):
    BLK = 2048
    grid = (5,)

    def body(agg_ref, cnt_ref, x_ref, wl_ref, wr_ref, b_ref, o_ref):
        aggs = agg_ref[0] + agg_ref[1]
        c = cnt_ref[0] + cnt_ref[1]
        inv = 1.0 / jnp.maximum(c, 1.0)
        a = aggs * inv[:, None]
        r = (jnp.dot(a, wl_ref[...], preferred_element_type=jnp.float32)
             + jnp.dot(x_ref[...], wr_ref[...],
                       preferred_element_type=jnp.float32)
             + b_ref[...])
        if relu:
            r = jnp.maximum(r, 0.0)
        o_ref[...] = r

    return pl.pallas_call(
        body,
        grid=grid,
        in_specs=[
            pl.BlockSpec((NC, BLK, D), lambda i: (0, i, 0)),
            pl.BlockSpec((NC, BLK), lambda i: (0, i)),
            pl.BlockSpec((BLK, D), lambda i: (i, 0)),
            pl.BlockSpec((D, D), lambda i: (0, 0)),
            pl.BlockSpec((D, D), lambda i: (0, 0)),
            pl.BlockSpec((1, D), lambda i: (0, 0)),
        ],
        out_specs=pl.BlockSpec((BLK, D), lambda i: (i, 0)),
        out_shape=jax.ShapeDtypeStruct((N_NODES, D), jnp.float32),
    )(agg, cnt, xin, wl_t, wr_t, b)


def kernel(x, edge_index, W1l, W1r, b1, W2l, W2r, b2):
    ei = edge_index.astype(jnp.int32)
    pad = EPAD - N_EDGES
    src = jnp.concatenate([ei[0], jnp.zeros((pad,), jnp.int32)])
    dst = jnp.concatenate(
        [ei[1], (jnp.arange(pad, dtype=jnp.int32) % 16) + N_NODES])
    dsts_cnt = dst.reshape(NW, KCH, CHUNK)

    kw = [K0 if (w % NC) == 0 else K1 for w in range(NW)]
    offs = [0]
    for w in range(NW - 1):
        offs.append(offs[-1] + kw[w])
    src_ch = src.reshape(EPAD // CHUNK, CHUNK)
    dst_ch = dst.reshape(EPAD // CHUNK, CHUNK)
    srcs = jnp.zeros((NW, KMAX, CHUNK), jnp.int32)
    dsts = jnp.full((NW, KMAX, CHUNK), N_NODES, jnp.int32)
    for w in range(NW):
        srcs = srcs.at[w, :kw[w]].set(src_ch[offs[w]:offs[w] + kw[w]])
        dsts = dsts.at[w, :kw[w]].set(dst_ch[offs[w]:offs[w] + kw[w]])
    zrows = jnp.zeros((SROWS, D), jnp.float32)

    ones_h = jnp.ones((CHUNK, D), jnp.float32)
    cntw = _cnt_kernel(dsts_cnt, ones_h, zrows)
    cnt = cntw.reshape(NC, NPAD, D)[:, :, 0]
    agg1 = _agg_plain(x, srcs, dsts, zrows)
    h = _combine(agg1, cnt, x, W1l.T, W1r.T, b1.reshape(1, D), relu=True)
    agg2 = _agg_plain(h, srcs, dsts, zrows)
    out = _combine(agg2, cnt, h, W2l.T, W2r.T, b2.reshape(1, D), relu=False)
    return out


# speedup vs baseline: 1.1303x; 1.0919x over previous
import functools

import jax
import jax.numpy as jnp
from jax import lax
from jax.experimental import pallas as pl
from jax.experimental.pallas import tpu as pltpu
from jax.experimental.pallas import tpu_sc as plsc

N_NODES = 10000
D = 128
DA = D + 16
N_EDGES = 320000

NC = 2
NS = 16
NW = NC * NS
CHUNK = 128
KCH = 80
GRP = 8
EPW = KCH * CHUNK
EPAD = NW * EPW
K0 = 80
K1 = 80
KMAX = max(K0, K1)
NPAD = 10240
RPS = NPAD // NS
SROWS = 128


def _make_agg(width):
    mesh = plsc.VectorSubcoreMesh(core_axis_name="c", subcore_axis_name="s")

    @functools.partial(
        pl.kernel, mesh=mesh,
        out_type=jax.ShapeDtypeStruct((NC, NPAD, width), jnp.float32),
        scratch_types=[
            pltpu.VMEM((GRP, CHUNK), jnp.int32),
            pltpu.VMEM((GRP, CHUNK), jnp.int32),
            pltpu.VMEM((CHUNK, width), jnp.float32),
            pltpu.VMEM((CHUNK, width), jnp.float32),
            pltpu.VMEM_SHARED((NPAD, width), jnp.float32),
            pltpu.SemaphoreType.DMA,
            pltpu.SemaphoreType.DMA,
            pltpu.SemaphoreType.DMA,
            pltpu.SemaphoreType.DMA,
        ])
    def k(table, srcs, dsts, zrows, agg_out, src_v, dst_v, rows_a, rows_b,
          agg_sh, sem_a, sem_b, sem_c, sem_d):
        rows_v = rows_a
        sid = lax.axis_index("s")
        cid = lax.axis_index("c")
        wid = sid * NC + cid

        pltpu.sync_copy(zrows, rows_v)
        for t in range(RPS // SROWS):
            pltpu.sync_copy(
                rows_v, agg_sh.at[pl.ds(sid * RPS + t * SROWS, SROWS)])
        plsc.subcore_barrier()

        bufs = (rows_a, rows_b)
        sems = ((sem_a, sem_c), (sem_b, sem_d))
        HC = CHUNK // 2
        ngrp = jnp.where(cid == 0, K0 // GRP, K1 // GRP)

        def start_gather(j, p):
            h0 = pltpu.async_copy(table.at[src_v.at[j].at[pl.ds(0, HC)]],
                                  bufs[p].at[pl.ds(0, HC)], sems[p][0])
            h1 = pltpu.async_copy(table.at[src_v.at[j].at[pl.ds(HC, HC)]],
                                  bufs[p].at[pl.ds(HC, HC)], sems[p][1])
            return (h0, h1)

        def group(g, _):
            pltpu.sync_copy(srcs.at[wid].at[pl.ds(g * GRP, GRP)], src_v)
            pltpu.sync_copy(dsts.at[wid].at[pl.ds(g * GRP, GRP)], dst_v)

            pend = [None, None]
            pend[0] = start_gather(0, 0)
            for j in range(GRP):
                p = j % 2
                if j + 1 < GRP:
                    pend[1 - p] = start_gather(j + 1, 1 - p)
                pend[p][0].wait()
                pend[p][1].wait()
                pltpu.sync_copy(bufs[p], agg_sh.at[dst_v.at[j]], add=True)
            return 0

        lax.fori_loop(0, ngrp, group, 0)
        plsc.subcore_barrier()

        for t in range(RPS // SROWS):
            off = sid * RPS + t * SROWS
            pltpu.sync_copy(agg_sh.at[pl.ds(off, SROWS)], rows_v)
            pltpu.sync_copy(rows_v, agg_out.at[cid].at[pl.ds(off, SROWS)])

    return k


_agg_plain = _make_agg(D)


def _make_cnt():
    mesh = plsc.VectorSubcoreMesh(core_axis_name="c", subcore_axis_name="s")

    @functools.partial(
        pl.kernel, mesh=mesh,
        out_type=jax.ShapeDtypeStruct((NC * NPAD, D), jnp.float32),
        scratch_types=[
            pltpu.VMEM((GRP, CHUNK), jnp.int32),
            pltpu.VMEM((CHUNK, D), jnp.float32),
            pltpu.VMEM((SROWS, D), jnp.float32),
            pltpu.VMEM_SHARED((NPAD, D), jnp.float32),
        ])
    def k(dsts, ones_h, zc, cnt_out, dst_v, ones_v, stage, cnt_sh):
        sid = lax.axis_index("s")
        cid = lax.axis_index("c")
        wid = sid * NC + cid

        pltpu.sync_copy(zc, stage)
        for t in range(RPS // SROWS):
            pltpu.sync_copy(
                stage, cnt_sh.at[pl.ds(sid * RPS + t * SROWS, SROWS)])
        pltpu.sync_copy(ones_h, ones_v)
        plsc.subcore_barrier()

        def group(g, _):
            pltpu.sync_copy(dsts.at[wid].at[pl.ds(g * GRP, GRP)], dst_v)

            def chunk(j, _):
                pltpu.sync_copy(ones_v, cnt_sh.at[dst_v.at[j]], add=True)
                return 0

            lax.fori_loop(0, GRP, chunk, 0)
            return 0

        lax.fori_loop(0, KCH // GRP, group, 0)
        plsc.subcore_barrier()

        for t in range(RPS // SROWS):
            off = sid * RPS + t * SROWS
            pltpu.sync_copy(cnt_sh.at[pl.ds(off, SROWS)], stage)
            pltpu.sync_copy(stage, cnt_out.at[pl.ds(cid * NPAD + off, SROWS)])

    return k


_cnt_kernel = _make_cnt()


def _combine(agg, cnt, xin, wl_t, wr_t, b, relu):
    BLK = 2048
    grid = (5,)

    def body(agg_ref, cnt_ref, x_ref, wl_ref, wr_ref, b_ref, o_ref):
        aggs = agg_ref[0] + agg_ref[1]
        c = cnt_ref[0] + cnt_ref[1]
        inv = 1.0 / jnp.maximum(c, 1.0)
        a = aggs * inv[:, None]
        r = (jnp.dot(a, wl_ref[...], preferred_element_type=jnp.float32)
             + jnp.dot(x_ref[...], wr_ref[...],
                       preferred_element_type=jnp.float32)
             + b_ref[...])
        if relu:
            r = jnp.maximum(r, 0.0)
        o_ref[...] = r

    return pl.pallas_call(
        body,
        grid=grid,
        in_specs=[
            pl.BlockSpec((NC, BLK, D), lambda i: (0, i, 0)),
            pl.BlockSpec((NC, BLK), lambda i: (0, i)),
            pl.BlockSpec((BLK, D), lambda i: (i, 0)),
            pl.BlockSpec((D, D), lambda i: (0, 0)),
            pl.BlockSpec((D, D), lambda i: (0, 0)),
            pl.BlockSpec((1, D), lambda i: (0, 0)),
        ],
        out_specs=pl.BlockSpec((BLK, D), lambda i: (i, 0)),
        out_shape=jax.ShapeDtypeStruct((N_NODES, D), jnp.float32),
    )(agg, cnt, xin, wl_t, wr_t, b)


def kernel(x, edge_index, W1l, W1r, b1, W2l, W2r, b2):
    ei = edge_index.astype(jnp.int32)
    pad = EPAD - N_EDGES
    src = jnp.concatenate([ei[0], jnp.zeros((pad,), jnp.int32)])
    dst = jnp.concatenate(
        [ei[1], (jnp.arange(pad, dtype=jnp.int32) % 16) + N_NODES])
    dsts_cnt = dst.reshape(NW, KCH, CHUNK)

    kw = [K0 if (w % NC) == 0 else K1 for w in range(NW)]
    offs = [0]
    for w in range(NW - 1):
        offs.append(offs[-1] + kw[w])
    src_ch = src.reshape(EPAD // CHUNK, CHUNK)
    dst_ch = dst.reshape(EPAD // CHUNK, CHUNK)
    srcs = jnp.zeros((NW, KMAX, CHUNK), jnp.int32)
    dsts = jnp.full((NW, KMAX, CHUNK), N_NODES, jnp.int32)
    for w in range(NW):
        srcs = srcs.at[w, :kw[w]].set(src_ch[offs[w]:offs[w] + kw[w]])
        dsts = dsts.at[w, :kw[w]].set(dst_ch[offs[w]:offs[w] + kw[w]])
    zrows = jnp.zeros((SROWS, D), jnp.float32)

    ones_h = jnp.ones((CHUNK, D), jnp.float32)
    cntw = _cnt_kernel(dsts_cnt, ones_h, zrows)
    cnt = cntw.reshape(NC, NPAD, D)[:, :, 0]
    agg1 = _agg_plain(x, srcs, dsts, zrows)
    h = _combine(agg1, cnt, x, W1l.T, W1r.T, b1.reshape(1, D), relu=True)
    agg2 = _agg_plain(h, srcs, dsts, zrows)
    out = _combine(agg2, cnt, h, W2l.T, W2r.T, b2.reshape(1, D), relu=False)
    return out


# speedup vs baseline: 1.1436x; 1.0118x over previous
import functools

import jax
import jax.numpy as jnp
from jax import lax
from jax.experimental import pallas as pl
from jax.experimental.pallas import tpu as pltpu
from jax.experimental.pallas import tpu_sc as plsc

N_NODES = 10000
D = 128
DA = D + 16
N_EDGES = 320000

NC = 2
NS = 16
NW = NC * NS
CHUNK = 128
KCH = 80
GRP = 8
EPW = KCH * CHUNK
EPAD = NW * EPW
K0 = 80
K1 = 80
KMAX = max(K0, K1)
NPAD = 10240
RPS = NPAD // NS
SROWS = 128


def _make_agg(width):
    mesh = plsc.VectorSubcoreMesh(core_axis_name="c", subcore_axis_name="s")

    @functools.partial(
        pl.kernel, mesh=mesh,
        out_type=jax.ShapeDtypeStruct((NC, NPAD, width), jnp.float32),
        scratch_types=[
            pltpu.VMEM((GRP, CHUNK), jnp.int32),
            pltpu.VMEM((GRP, CHUNK), jnp.int32),
            pltpu.VMEM((CHUNK, width), jnp.float32),
            pltpu.VMEM((CHUNK, width), jnp.float32),
            pltpu.VMEM_SHARED((NPAD, width), jnp.float32),
            pltpu.SemaphoreType.DMA,
            pltpu.SemaphoreType.DMA,
            pltpu.SemaphoreType.DMA,
            pltpu.SemaphoreType.DMA,
            pltpu.SemaphoreType.DMA,
            pltpu.SemaphoreType.DMA,
            pltpu.SemaphoreType.DMA,
            pltpu.SemaphoreType.DMA,
        ])
    def k(table, srcs, dsts, zrows, agg_out, src_v, dst_v, rows_a, rows_b,
          agg_sh, *sems8):
        rows_v = rows_a
        sid = lax.axis_index("s")
        cid = lax.axis_index("c")
        wid = sid * NC + cid

        pltpu.sync_copy(zrows, rows_v)
        for t in range(RPS // SROWS):
            pltpu.sync_copy(
                rows_v, agg_sh.at[pl.ds(sid * RPS + t * SROWS, SROWS)])
        plsc.subcore_barrier()

        bufs = (rows_a, rows_b)
        NSPL = 4
        sems = (sems8[:NSPL], sems8[NSPL:])
        HC = CHUNK // NSPL
        ngrp = jnp.where(cid == 0, K0 // GRP, K1 // GRP)

        def start_gather(j, p):
            return tuple(
                pltpu.async_copy(
                    table.at[src_v.at[j].at[pl.ds(q * HC, HC)]],
                    bufs[p].at[pl.ds(q * HC, HC)], sems[p][q])
                for q in range(NSPL))

        def group(g, _):
            pltpu.sync_copy(srcs.at[wid].at[pl.ds(g * GRP, GRP)], src_v)
            pltpu.sync_copy(dsts.at[wid].at[pl.ds(g * GRP, GRP)], dst_v)

            pend = [None, None]
            pend[0] = start_gather(0, 0)
            for j in range(GRP):
                p = j % 2
                if j + 1 < GRP:
                    pend[1 - p] = start_gather(j + 1, 1 - p)
                for h in pend[p]:
                    h.wait()
                pltpu.sync_copy(bufs[p], agg_sh.at[dst_v.at[j]], add=True)
            return 0

        lax.fori_loop(0, ngrp, group, 0)
        plsc.subcore_barrier()

        for t in range(RPS // SROWS):
            off = sid * RPS + t * SROWS
            pltpu.sync_copy(agg_sh.at[pl.ds(off, SROWS)], rows_v)
            pltpu.sync_copy(rows_v, agg_out.at[cid].at[pl.ds(off, SROWS)])

    return k


_agg_plain = _make_agg(D)


def _make_cnt():
    mesh = plsc.VectorSubcoreMesh(core_axis_name="c", subcore_axis_name="s")

    @functools.partial(
        pl.kernel, mesh=mesh,
        out_type=jax.ShapeDtypeStruct((NC * NPAD, D), jnp.float32),
        scratch_types=[
            pltpu.VMEM((GRP, CHUNK), jnp.int32),
            pltpu.VMEM((CHUNK, D), jnp.float32),
            pltpu.VMEM((SROWS, D), jnp.float32),
            pltpu.VMEM_SHARED((NPAD, D), jnp.float32),
        ])
    def k(dsts, ones_h, zc, cnt_out, dst_v, ones_v, stage, cnt_sh):
        sid = lax.axis_index("s")
        cid = lax.axis_index("c")
        wid = sid * NC + cid

        pltpu.sync_copy(zc, stage)
        for t in range(RPS // SROWS):
            pltpu.sync_copy(
                stage, cnt_sh.at[pl.ds(sid * RPS + t * SROWS, SROWS)])
        pltpu.sync_copy(ones_h, ones_v)
        plsc.subcore_barrier()

        def group(g, _):
            pltpu.sync_copy(dsts.at[wid].at[pl.ds(g * GRP, GRP)], dst_v)

            def chunk(j, _):
                pltpu.sync_copy(ones_v, cnt_sh.at[dst_v.at[j]], add=True)
                return 0

            lax.fori_loop(0, GRP, chunk, 0)
            return 0

        lax.fori_loop(0, KCH // GRP, group, 0)
        plsc.subcore_barrier()

        for t in range(RPS // SROWS):
            off = sid * RPS + t * SROWS
            pltpu.sync_copy(cnt_sh.at[pl.ds(off, SROWS)], stage)
            pltpu.sync_copy(stage, cnt_out.at[pl.ds(cid * NPAD + off, SROWS)])

    return k


_cnt_kernel = _make_cnt()


def _combine(agg, cnt, xin, wl_t, wr_t, b, relu):
    BLK = 2048
    grid = (5,)

    def body(agg_ref, cnt_ref, x_ref, wl_ref, wr_ref, b_ref, o_ref):
        aggs = agg_ref[0] + agg_ref[1]
        c = cnt_ref[0] + cnt_ref[1]
        inv = 1.0 / jnp.maximum(c, 1.0)
        a = aggs * inv[:, None]
        r = (jnp.dot(a, wl_ref[...], preferred_element_type=jnp.float32)
             + jnp.dot(x_ref[...], wr_ref[...],
                       preferred_element_type=jnp.float32)
             + b_ref[...])
        if relu:
            r = jnp.maximum(r, 0.0)
        o_ref[...] = r

    return pl.pallas_call(
        body,
        grid=grid,
        in_specs=[
            pl.BlockSpec((NC, BLK, D), lambda i: (0, i, 0)),
            pl.BlockSpec((NC, BLK), lambda i: (0, i)),
            pl.BlockSpec((BLK, D), lambda i: (i, 0)),
            pl.BlockSpec((D, D), lambda i: (0, 0)),
            pl.BlockSpec((D, D), lambda i: (0, 0)),
            pl.BlockSpec((1, D), lambda i: (0, 0)),
        ],
        out_specs=pl.BlockSpec((BLK, D), lambda i: (i, 0)),
        out_shape=jax.ShapeDtypeStruct((N_NODES, D), jnp.float32),
    )(agg, cnt, xin, wl_t, wr_t, b)


def kernel(x, edge_index, W1l, W1r, b1, W2l, W2r, b2):
    ei = edge_index.astype(jnp.int32)
    pad = EPAD - N_EDGES
    src = jnp.concatenate([ei[0], jnp.zeros((pad,), jnp.int32)])
    dst = jnp.concatenate(
        [ei[1], (jnp.arange(pad, dtype=jnp.int32) % 16) + N_NODES])
    dsts_cnt = dst.reshape(NW, KCH, CHUNK)

    kw = [K0 if (w % NC) == 0 else K1 for w in range(NW)]
    offs = [0]
    for w in range(NW - 1):
        offs.append(offs[-1] + kw[w])
    src_ch = src.reshape(EPAD // CHUNK, CHUNK)
    dst_ch = dst.reshape(EPAD // CHUNK, CHUNK)
    srcs = jnp.zeros((NW, KMAX, CHUNK), jnp.int32)
    dsts = jnp.full((NW, KMAX, CHUNK), N_NODES, jnp.int32)
    for w in range(NW):
        srcs = srcs.at[w, :kw[w]].set(src_ch[offs[w]:offs[w] + kw[w]])
        dsts = dsts.at[w, :kw[w]].set(dst_ch[offs[w]:offs[w] + kw[w]])
    zrows = jnp.zeros((SROWS, D), jnp.float32)

    ones_h = jnp.ones((CHUNK, D), jnp.float32)
    cntw = _cnt_kernel(dsts_cnt, ones_h, zrows)
    cnt = cntw.reshape(NC, NPAD, D)[:, :, 0]
    agg1 = _agg_plain(x, srcs, dsts, zrows)
    h = _combine(agg1, cnt, x, W1l.T, W1r.T, b1.reshape(1, D), relu=True)
    agg2 = _agg_plain(h, srcs, dsts, zrows)
    out = _combine(agg2, cnt, h, W2l.T, W2r.T, b2.reshape(1, D), relu=False)
    return out


# speedup vs baseline: 1.2717x; 1.1120x over previous
import functools

import jax
import jax.numpy as jnp
from jax import lax
from jax.experimental import pallas as pl
from jax.experimental.pallas import tpu as pltpu
from jax.experimental.pallas import tpu_sc as plsc

N_NODES = 10000
D = 128
DA = D + 16
N_EDGES = 320000

NC = 2
NS = 16
NW = NC * NS
CHUNK = 128
KCH = 80
GRP = 8
EPW = KCH * CHUNK
EPAD = NW * EPW
K0 = 112
K1 = 48
KMAX = max(K0, K1)
NPAD = 10240
RPS = NPAD // NS
SROWS = 128


def _make_agg(width):
    mesh = plsc.VectorSubcoreMesh(core_axis_name="c", subcore_axis_name="s")

    @functools.partial(
        pl.kernel, mesh=mesh,
        out_type=jax.ShapeDtypeStruct((NC, NPAD, width), jnp.float32),
        scratch_types=[
            pltpu.VMEM((GRP, CHUNK), jnp.int32),
            pltpu.VMEM((GRP, CHUNK), jnp.int32),
            pltpu.VMEM((CHUNK, width), jnp.float32),
            pltpu.VMEM((CHUNK, width), jnp.float32),
            pltpu.VMEM_SHARED((NPAD, width), jnp.float32),
            pltpu.SemaphoreType.DMA,
            pltpu.SemaphoreType.DMA,
            pltpu.SemaphoreType.DMA,
            pltpu.SemaphoreType.DMA,
            pltpu.SemaphoreType.DMA,
            pltpu.SemaphoreType.DMA,
            pltpu.SemaphoreType.DMA,
            pltpu.SemaphoreType.DMA,
        ])
    def k(table, srcs, dsts, zrows, agg_out, src_v, dst_v, rows_a, rows_b,
          agg_sh, *sems8):
        rows_v = rows_a
        sid = lax.axis_index("s")
        cid = lax.axis_index("c")
        wid = sid * NC + cid

        pltpu.sync_copy(zrows, rows_v)
        for t in range(RPS // SROWS):
            pltpu.sync_copy(
                rows_v, agg_sh.at[pl.ds(sid * RPS + t * SROWS, SROWS)])
        plsc.subcore_barrier()

        bufs = (rows_a, rows_b)
        NSPL = 4
        sems = (sems8[:NSPL], sems8[NSPL:])
        HC = CHUNK // NSPL
        ngrp = jnp.where(cid == 0, K0 // GRP, K1 // GRP)

        def start_gather(j, p):
            return tuple(
                pltpu.async_copy(
                    table.at[src_v.at[j].at[pl.ds(q * HC, HC)]],
                    bufs[p].at[pl.ds(q * HC, HC)], sems[p][q])
                for q in range(NSPL))

        def group(g, _):
            pltpu.sync_copy(srcs.at[wid].at[pl.ds(g * GRP, GRP)], src_v)
            pltpu.sync_copy(dsts.at[wid].at[pl.ds(g * GRP, GRP)], dst_v)

            pend = [None, None]
            pend[0] = start_gather(0, 0)
            for j in range(GRP):
                p = j % 2
                if j + 1 < GRP:
                    pend[1 - p] = start_gather(j + 1, 1 - p)
                for h in pend[p]:
                    h.wait()
                pltpu.sync_copy(bufs[p], agg_sh.at[dst_v.at[j]], add=True)
            return 0

        lax.fori_loop(0, ngrp, group, 0)
        plsc.subcore_barrier()

        for t in range(RPS // SROWS):
            off = sid * RPS + t * SROWS
            pltpu.sync_copy(agg_sh.at[pl.ds(off, SROWS)], rows_v)
            pltpu.sync_copy(rows_v, agg_out.at[cid].at[pl.ds(off, SROWS)])

    return k


_agg_plain = _make_agg(D)


def _make_cnt():
    mesh = plsc.VectorSubcoreMesh(core_axis_name="c", subcore_axis_name="s")

    @functools.partial(
        pl.kernel, mesh=mesh,
        out_type=jax.ShapeDtypeStruct((NC * NPAD, D), jnp.float32),
        scratch_types=[
            pltpu.VMEM((GRP, CHUNK), jnp.int32),
            pltpu.VMEM((CHUNK, D), jnp.float32),
            pltpu.VMEM((SROWS, D), jnp.float32),
            pltpu.VMEM_SHARED((NPAD, D), jnp.float32),
        ])
    def k(dsts, ones_h, zc, cnt_out, dst_v, ones_v, stage, cnt_sh):
        sid = lax.axis_index("s")
        cid = lax.axis_index("c")
        wid = sid * NC + cid

        pltpu.sync_copy(zc, stage)
        for t in range(RPS // SROWS):
            pltpu.sync_copy(
                stage, cnt_sh.at[pl.ds(sid * RPS + t * SROWS, SROWS)])
        pltpu.sync_copy(ones_h, ones_v)
        plsc.subcore_barrier()

        def group(g, _):
            pltpu.sync_copy(dsts.at[wid].at[pl.ds(g * GRP, GRP)], dst_v)

            def chunk(j, _):
                pltpu.sync_copy(ones_v, cnt_sh.at[dst_v.at[j]], add=True)
                return 0

            lax.fori_loop(0, GRP, chunk, 0)
            return 0

        lax.fori_loop(0, KCH // GRP, group, 0)
        plsc.subcore_barrier()

        for t in range(RPS // SROWS):
            off = sid * RPS + t * SROWS
            pltpu.sync_copy(cnt_sh.at[pl.ds(off, SROWS)], stage)
            pltpu.sync_copy(stage, cnt_out.at[pl.ds(cid * NPAD + off, SROWS)])

    return k


_cnt_kernel = _make_cnt()


def _combine(agg, cnt, xin, wl_t, wr_t, b, relu):
    BLK = 2048
    grid = (5,)

    def body(agg_ref, cnt_ref, x_ref, wl_ref, wr_ref, b_ref, o_ref):
        aggs = agg_ref[0] + agg_ref[1]
        c = cnt_ref[0] + cnt_ref[1]
        inv = 1.0 / jnp.maximum(c, 1.0)
        a = aggs * inv[:, None]
        r = (jnp.dot(a, wl_ref[...], preferred_element_type=jnp.float32)
             + jnp.dot(x_ref[...], wr_ref[...],
                       preferred_element_type=jnp.float32)
             + b_ref[...])
        if relu:
            r = jnp.maximum(r, 0.0)
        o_ref[...] = r

    return pl.pallas_call(
        body,
        grid=grid,
        in_specs=[
            pl.BlockSpec((NC, BLK, D), lambda i: (0, i, 0)),
            pl.BlockSpec((NC, BLK), lambda i: (0, i)),
            pl.BlockSpec((BLK, D), lambda i: (i, 0)),
            pl.BlockSpec((D, D), lambda i: (0, 0)),
            pl.BlockSpec((D, D), lambda i: (0, 0)),
            pl.BlockSpec((1, D), lambda i: (0, 0)),
        ],
        out_specs=pl.BlockSpec((BLK, D), lambda i: (i, 0)),
        out_shape=jax.ShapeDtypeStruct((N_NODES, D), jnp.float32),
    )(agg, cnt, xin, wl_t, wr_t, b)


def kernel(x, edge_index, W1l, W1r, b1, W2l, W2r, b2):
    ei = edge_index.astype(jnp.int32)
    pad = EPAD - N_EDGES
    src = jnp.concatenate([ei[0], jnp.zeros((pad,), jnp.int32)])
    dst = jnp.concatenate(
        [ei[1], (jnp.arange(pad, dtype=jnp.int32) % 16) + N_NODES])
    dsts_cnt = dst.reshape(NW, KCH, CHUNK)

    kw = [K0 if (w % NC) == 0 else K1 for w in range(NW)]
    offs = [0]
    for w in range(NW - 1):
        offs.append(offs[-1] + kw[w])
    src_ch = src.reshape(EPAD // CHUNK, CHUNK)
    dst_ch = dst.reshape(EPAD // CHUNK, CHUNK)
    srcs = jnp.zeros((NW, KMAX, CHUNK), jnp.int32)
    dsts = jnp.full((NW, KMAX, CHUNK), N_NODES, jnp.int32)
    for w in range(NW):
        srcs = srcs.at[w, :kw[w]].set(src_ch[offs[w]:offs[w] + kw[w]])
        dsts = dsts.at[w, :kw[w]].set(dst_ch[offs[w]:offs[w] + kw[w]])
    zrows = jnp.zeros((SROWS, D), jnp.float32)

    ones_h = jnp.ones((CHUNK, D), jnp.float32)
    cntw = _cnt_kernel(dsts_cnt, ones_h, zrows)
    cnt = cntw.reshape(NC, NPAD, D)[:, :, 0]
    agg1 = _agg_plain(x, srcs, dsts, zrows)
    h = _combine(agg1, cnt, x, W1l.T, W1r.T, b1.reshape(1, D), relu=True)
    agg2 = _agg_plain(h, srcs, dsts, zrows)
    out = _combine(agg2, cnt, h, W2l.T, W2r.T, b2.reshape(1, D), relu=False)
    return out


# speedup vs baseline: 1.2815x; 1.0077x over previous
import functools

import jax
import jax.numpy as jnp
from jax import lax
from jax.experimental import pallas as pl
from jax.experimental.pallas import tpu as pltpu
from jax.experimental.pallas import tpu_sc as plsc

N_NODES = 10000
D = 128
DA = D + 16
N_EDGES = 320000

NC = 2
NS = 16
NW = NC * NS
CHUNK = 128
KCH = 80
GRP = 8
EPW = KCH * CHUNK
EPAD = NW * EPW
K0 = 128
K1 = 32
KMAX = max(K0, K1)
NPAD = 10240
RPS = NPAD // NS
SROWS = 128


def _make_agg(width):
    mesh = plsc.VectorSubcoreMesh(core_axis_name="c", subcore_axis_name="s")

    @functools.partial(
        pl.kernel, mesh=mesh,
        out_type=jax.ShapeDtypeStruct((NC, NPAD, width), jnp.float32),
        scratch_types=[
            pltpu.VMEM((GRP, CHUNK), jnp.int32),
            pltpu.VMEM((GRP, CHUNK), jnp.int32),
            pltpu.VMEM((CHUNK, width), jnp.float32),
            pltpu.VMEM((CHUNK, width), jnp.float32),
            pltpu.VMEM_SHARED((NPAD, width), jnp.float32),
            pltpu.SemaphoreType.DMA,
            pltpu.SemaphoreType.DMA,
            pltpu.SemaphoreType.DMA,
            pltpu.SemaphoreType.DMA,
            pltpu.SemaphoreType.DMA,
            pltpu.SemaphoreType.DMA,
            pltpu.SemaphoreType.DMA,
            pltpu.SemaphoreType.DMA,
        ])
    def k(table, srcs, dsts, zrows, agg_out, src_v, dst_v, rows_a, rows_b,
          agg_sh, *sems8):
        rows_v = rows_a
        sid = lax.axis_index("s")
        cid = lax.axis_index("c")
        wid = sid * NC + cid

        pltpu.sync_copy(zrows, rows_v)
        for t in range(RPS // SROWS):
            pltpu.sync_copy(
                rows_v, agg_sh.at[pl.ds(sid * RPS + t * SROWS, SROWS)])
        plsc.subcore_barrier()

        bufs = (rows_a, rows_b)
        NSPL = 4
        sems = (sems8[:NSPL], sems8[NSPL:])
        HC = CHUNK // NSPL
        ngrp = jnp.where(cid == 0, K0 // GRP, K1 // GRP)

        def start_gather(j, p):
            return tuple(
                pltpu.async_copy(
                    table.at[src_v.at[j].at[pl.ds(q * HC, HC)]],
                    bufs[p].at[pl.ds(q * HC, HC)], sems[p][q])
                for q in range(NSPL))

        def group(g, _):
            pltpu.sync_copy(srcs.at[wid].at[pl.ds(g * GRP, GRP)], src_v)
            pltpu.sync_copy(dsts.at[wid].at[pl.ds(g * GRP, GRP)], dst_v)

            pend = [None, None]
            pend[0] = start_gather(0, 0)
            for j in range(GRP):
                p = j % 2
                if j + 1 < GRP:
                    pend[1 - p] = start_gather(j + 1, 1 - p)
                for h in pend[p]:
                    h.wait()
                pltpu.sync_copy(bufs[p], agg_sh.at[dst_v.at[j]], add=True)
            return 0

        lax.fori_loop(0, ngrp, group, 0)
        plsc.subcore_barrier()

        for t in range(RPS // SROWS):
            off = sid * RPS + t * SROWS
            pltpu.sync_copy(agg_sh.at[pl.ds(off, SROWS)], rows_v)
            pltpu.sync_copy(rows_v, agg_out.at[cid].at[pl.ds(off, SROWS)])

    return k


_agg_plain = _make_agg(D)


def _make_cnt():
    mesh = plsc.VectorSubcoreMesh(core_axis_name="c", subcore_axis_name="s")

    @functools.partial(
        pl.kernel, mesh=mesh,
        out_type=jax.ShapeDtypeStruct((NC * NPAD, D), jnp.float32),
        scratch_types=[
            pltpu.VMEM((GRP, CHUNK), jnp.int32),
            pltpu.VMEM((CHUNK, D), jnp.float32),
            pltpu.VMEM((SROWS, D), jnp.float32),
            pltpu.VMEM_SHARED((NPAD, D), jnp.float32),
        ])
    def k(dsts, ones_h, zc, cnt_out, dst_v, ones_v, stage, cnt_sh):
        sid = lax.axis_index("s")
        cid = lax.axis_index("c")
        wid = sid * NC + cid

        pltpu.sync_copy(zc, stage)
        for t in range(RPS // SROWS):
            pltpu.sync_copy(
                stage, cnt_sh.at[pl.ds(sid * RPS + t * SROWS, SROWS)])
        pltpu.sync_copy(ones_h, ones_v)
        plsc.subcore_barrier()

        def group(g, _):
            pltpu.sync_copy(dsts.at[wid].at[pl.ds(g * GRP, GRP)], dst_v)

            def chunk(j, _):
                pltpu.sync_copy(ones_v, cnt_sh.at[dst_v.at[j]], add=True)
                return 0

            lax.fori_loop(0, GRP, chunk, 0)
            return 0

        lax.fori_loop(0, KCH // GRP, group, 0)
        plsc.subcore_barrier()

        for t in range(RPS // SROWS):
            off = sid * RPS + t * SROWS
            pltpu.sync_copy(cnt_sh.at[pl.ds(off, SROWS)], stage)
            pltpu.sync_copy(stage, cnt_out.at[pl.ds(cid * NPAD + off, SROWS)])

    return k


_cnt_kernel = _make_cnt()


def _combine(agg, cnt, xin, wl_t, wr_t, b, relu):
    BLK = 2048
    grid = (5,)

    def body(agg_ref, cnt_ref, x_ref, wl_ref, wr_ref, b_ref, o_ref):
        aggs = agg_ref[0] + agg_ref[1]
        c = cnt_ref[0] + cnt_ref[1]
        inv = 1.0 / jnp.maximum(c, 1.0)
        a = aggs * inv[:, None]
        r = (jnp.dot(a, wl_ref[...], preferred_element_type=jnp.float32)
             + jnp.dot(x_ref[...], wr_ref[...],
                       preferred_element_type=jnp.float32)
             + b_ref[...])
        if relu:
            r = jnp.maximum(r, 0.0)
        o_ref[...] = r

    return pl.pallas_call(
        body,
        grid=grid,
        in_specs=[
            pl.BlockSpec((NC, BLK, D), lambda i: (0, i, 0)),
            pl.BlockSpec((NC, BLK), lambda i: (0, i)),
            pl.BlockSpec((BLK, D), lambda i: (i, 0)),
            pl.BlockSpec((D, D), lambda i: (0, 0)),
            pl.BlockSpec((D, D), lambda i: (0, 0)),
            pl.BlockSpec((1, D), lambda i: (0, 0)),
        ],
        out_specs=pl.BlockSpec((BLK, D), lambda i: (i, 0)),
        out_shape=jax.ShapeDtypeStruct((N_NODES, D), jnp.float32),
    )(agg, cnt, xin, wl_t, wr_t, b)


def kernel(x, edge_index, W1l, W1r, b1, W2l, W2r, b2):
    ei = edge_index.astype(jnp.int32)
    pad = EPAD - N_EDGES
    src = jnp.concatenate([ei[0], jnp.zeros((pad,), jnp.int32)])
    dst = jnp.concatenate(
        [ei[1], (jnp.arange(pad, dtype=jnp.int32) % 16) + N_NODES])
    dsts_cnt = dst.reshape(NW, KCH, CHUNK)

    kw = [K0 if (w % NC) == 0 else K1 for w in range(NW)]
    offs = [0]
    for w in range(NW - 1):
        offs.append(offs[-1] + kw[w])
    src_ch = src.reshape(EPAD // CHUNK, CHUNK)
    dst_ch = dst.reshape(EPAD // CHUNK, CHUNK)
    srcs = jnp.zeros((NW, KMAX, CHUNK), jnp.int32)
    dsts = jnp.full((NW, KMAX, CHUNK), N_NODES, jnp.int32)
    for w in range(NW):
        srcs = srcs.at[w, :kw[w]].set(src_ch[offs[w]:offs[w] + kw[w]])
        dsts = dsts.at[w, :kw[w]].set(dst_ch[offs[w]:offs[w] + kw[w]])
    zrows = jnp.zeros((SROWS, D), jnp.float32)

    ones_h = jnp.ones((CHUNK, D), jnp.float32)
    cntw = _cnt_kernel(dsts_cnt, ones_h, zrows)
    cnt = cntw.reshape(NC, NPAD, D)[:, :, 0]
    agg1 = _agg_plain(x, srcs, dsts, zrows)
    h = _combine(agg1, cnt, x, W1l.T, W1r.T, b1.reshape(1, D), relu=True)
    agg2 = _agg_plain(h, srcs, dsts, zrows)
    out = _combine(agg2, cnt, h, W2l.T, W2r.T, b2.reshape(1, D), relu=False)
    return out
